# baseline (speedup 1.0000x reference)
import functools

import jax
import jax.numpy as jnp
from jax import lax
from jax.experimental import pallas as pl
from jax.experimental.pallas import tpu as pltpu
from jax.experimental.pallas import tpu_sc as plsc

EMBED = 32
NC, NS = 2, 16
NW = NC * NS


@functools.lru_cache(maxsize=None)
def _make_gather(batch: int, hist: int):
    bw = batch // NW
    assert bw * NW == batch

    mesh = plsc.VectorSubcoreMesh(
        core_axis_name="c", subcore_axis_name="s",
        num_cores=NC, num_subcores=NS)

    @functools.partial(
        pl.kernel,
        out_type=jax.ShapeDtypeStruct((hist, EMBED, batch, 1), jnp.float32),
        mesh=mesh,
        scratch_types=[
            pltpu.VMEM((2, bw), jnp.int32),
            pltpu.VMEM((2, bw, EMBED), jnp.float32),
            pltpu.SemaphoreType.DMA,
            pltpu.SemaphoreType.DMA,
            pltpu.SemaphoreType.DMA,
            pltpu.SemaphoreType.DMA,
        ],
        compiler_params=pltpu.CompilerParams(use_tc_tiling_on_sc=False),
    )
    def grab(idx_hbm, tab_hbm, out_hbm, idx_v, rows_v, g0, g1, o0, o1):
        wid = lax.axis_index("s") * NC + lax.axis_index("c")
        b0 = wid * bw
        gsem = (g0, g1)
        osem = (o0, o1)

        def stage_and_gather(h):
            pltpu.sync_copy(idx_hbm.at[h, pl.ds(b0, bw)], idx_v.at[0])
            pltpu.async_copy(tab_hbm.at[idx_v.at[0]],
                             rows_v.at[0], gsem[0]).wait()

        def out_descs(h):
            return [
                pltpu.make_async_copy(rows_v.at[0, :, pl.ds(k, 1)],
                                      out_hbm.at[h, k, pl.ds(b0, bw)],
                                      osem[0])
                for k in range(EMBED)
            ]

        def body(h, carry):
            @pl.when(h > 0)
            def _():
                for d in out_descs(h - 1):
                    d.wait()
            stage_and_gather(h)
            for d in out_descs(h):
                d.start()
            return carry

        lax.fori_loop(0, hist, body, 0)
        for d in out_descs(hist - 1):
            d.wait()

    return grab


def kernel(inputs, embeddings):
    batch, hist = inputs.shape
    idx_t = inputs.T if inputs.dtype == jnp.int32 else inputs.T.astype(jnp.int32)
    out_t = _make_gather(batch, hist)(idx_t, embeddings)
    return out_t.reshape(hist, EMBED, batch).transpose(2, 0, 1)


# speedup vs baseline: 63.9395x; 63.9395x over previous
import functools

import jax
import jax.numpy as jnp
from jax import lax
from jax.experimental import pallas as pl
from jax.experimental.pallas import tpu as pltpu
from jax.experimental.pallas import tpu_sc as plsc

EMBED = 32
NC, NS = 2, 16
NW = NC * NS


@functools.lru_cache(maxsize=None)
def _make_gather(batch: int, hist: int):
    bw = batch // NW
    assert bw * NW == batch

    mesh = plsc.VectorSubcoreMesh(
        core_axis_name="c", subcore_axis_name="s",
        num_cores=NC, num_subcores=NS)

    @functools.partial(
        pl.kernel,
        out_type=jax.ShapeDtypeStruct((hist, batch, EMBED), jnp.float32),
        mesh=mesh,
        scratch_types=[
            pltpu.VMEM((hist, bw), jnp.int32),
            pltpu.VMEM((3, bw, EMBED), jnp.float32),
            pltpu.SemaphoreType.DMA,
            pltpu.SemaphoreType.DMA,
            pltpu.SemaphoreType.DMA,
            pltpu.SemaphoreType.DMA,
            pltpu.SemaphoreType.DMA,
            pltpu.SemaphoreType.DMA,
        ],
        compiler_params=pltpu.CompilerParams(use_tc_tiling_on_sc=False),
    )
    def grab(idx_hbm, tab_hbm, out_hbm, idx_v, rows_v,
             g0, g1, g2, o0, o1, o2):
        wid = lax.axis_index("s") * NC + lax.axis_index("c")
        b0 = wid * bw
        gsem = (g0, g1, g2)
        osem = (o0, o1, o2)
        NB = 3

        pltpu.sync_copy(idx_hbm.at[:, pl.ds(b0, bw)], idx_v)

        def fire_gather(h):
            return pltpu.async_copy(tab_hbm.at[idx_v.at[h]],
                                    rows_v.at[h % NB], gsem[h % NB])

        def fire_out(h):
            return pltpu.async_copy(rows_v.at[h % NB],
                                    out_hbm.at[h, pl.ds(b0, bw)],
                                    osem[h % NB])

        gds = {h: fire_gather(h) for h in range(NB - 1)}
        ods = {}
        for h in range(hist):
            if h + NB - 1 < hist:
                if h >= 1:
                    ods.pop(h - 1).wait()
                gds[h + NB - 1] = fire_gather(h + NB - 1)
            gds.pop(h).wait()
            ods[h] = fire_out(h)
        for d in ods.values():
            d.wait()

    return grab


def kernel(inputs, embeddings):
    batch, hist = inputs.shape
    idx_t = inputs.T if inputs.dtype == jnp.int32 else inputs.T.astype(jnp.int32)
    out_t = _make_gather(batch, hist)(idx_t, embeddings)
    return out_t.transpose(1, 0, 2)
